# Initial kernel scaffold; baseline (speedup 1.0000x reference)
import jax
import jax.numpy as jnp
from jax.experimental import pallas as pl


def kernel(edge_index, score, target_edges, num_nodes):
    raise NotImplementedError("write your pallas kernel here")


# speedup vs baseline: 13.0589x; 13.0589x over previous
import functools

import jax
import jax.numpy as jnp
from jax import lax
from jax.experimental import pallas as pl
from jax.experimental.pallas import tpu as pltpu
from jax.experimental.pallas import tpu_sc as plsc

NCORE = 2
NSUB = 16
L = 16

N_PAD = 51200
WINDOW = 6400
NROUND = N_PAD // WINDOW
STRIPE = WINDOW // NSUB
LOCAL_N = NROUND * STRIPE
EDGE_CH = 4000
TGT_CH = 3200


def _sqrt_pos(x):
    i = lax.bitcast_convert_type(x, jnp.int32)
    y = lax.bitcast_convert_type(jnp.int32(0x5F3759DF) - (i >> 1), jnp.float32)
    for _ in range(3):
        y = y * (1.5 - 0.5 * x * y * y)
    return jnp.where(x > 0.0, x * y, 0.0)


def _build(n_edges):
    e_per_w = n_edges // NSUB
    n_ech = e_per_w // EDGE_CH
    vpc = EDGE_CH // L
    n_tch = N_PAD // TGT_CH
    tvpc = TGT_CH // L
    assert e_per_w % EDGE_CH == 0

    mesh = plsc.VectorSubcoreMesh(core_axis_name="c", subcore_axis_name="s")

    @functools.partial(
        pl.kernel,
        mesh=mesh,
        out_type=jax.ShapeDtypeStruct((NCORE * NSUB, L), jnp.float32),
        compiler_params=pltpu.CompilerParams(
            needs_layout_passes=False, use_tc_tiling_on_sc=False),
        scratch_types=[
            pltpu.VMEM((N_PAD,), jnp.float32),
            pltpu.VMEM((N_PAD,), jnp.float32),
            pltpu.VMEM((EDGE_CH,), jnp.int32),
            pltpu.VMEM((EDGE_CH,), jnp.float32),
            pltpu.VMEM((LOCAL_N,), jnp.float32),
            pltpu.VMEM((LOCAL_N,), jnp.float32),
            pltpu.VMEM((NSUB, STRIPE), jnp.float32),
            pltpu.VMEM((L,), jnp.float32),
            pltpu.VMEM_SHARED((NSUB, WINDOW), jnp.float32),
        ],
    )
    def k(edge_hbm, score_hbm, tgt_hbm, out_hbm,
          acc_a, acc_q, idx_buf, val_buf, ones_buf, tsum_buf, rbuf, loss_buf,
          shared):
        cid = lax.axis_index("c")
        sid = lax.axis_index("s")

        zero16 = jnp.zeros((L,), jnp.float32)

        def zero_body(j, carry):
            acc_a[pl.ds(j * L, L)] = zero16
            acc_q[pl.ds(j * L, L)] = zero16
            return carry
        lax.fori_loop(0, N_PAD // L, zero_body, 0)

        ebase = sid * e_per_w

        def chunk_body(ci, carry):
            off = ebase + ci * EDGE_CH
            pltpu.sync_copy(edge_hbm.at[pl.ds(cid * n_edges + off, EDGE_CH)],
                            idx_buf)
            pltpu.sync_copy(score_hbm.at[pl.ds(off, EDGE_CH)], val_buf)

            def vec_body(j, c2):
                vi = idx_buf[pl.ds(j * L, L)]
                vs = val_buf[pl.ds(j * L, L)]
                plsc.addupdate_scatter(acc_a, [vi], vs)
                plsc.addupdate_scatter(acc_q, [vi], vs * vs)
                return c2
            lax.fori_loop(0, vpc, vec_body, 0)
            return carry
        lax.fori_loop(0, n_ech, chunk_body, 0)

        def zo_body(j, carry):
            ones_buf[pl.ds(j * L, L)] = zero16
            return carry
        lax.fori_loop(0, LOCAL_N // L, zo_body, 0)

        one16 = jnp.ones((L,), jnp.float32)

        def tchunk_body(ci, carry):
            pltpu.sync_copy(
                tgt_hbm.at[pl.ds(cid * N_PAD + ci * TGT_CH, TGT_CH)],
                idx_buf.at[pl.ds(0, TGT_CH)])

            def tvec_body(j, c2):
                v = idx_buf[pl.ds(j * L, L)]
                owner = (v % WINDOW) // STRIPE
                loc = (v // WINDOW) * STRIPE + v % STRIPE
                m = owner == sid
                plsc.store_scatter(ones_buf, [loc], one16, mask=m)
                return c2
            lax.fori_loop(0, tvpc, tvec_body, 0)
            return carry
        lax.fori_loop(0, n_tch, tchunk_body, 0)

        lv = jnp.zeros((L,), jnp.float32)
        spc = STRIPE // L
        for r in range(NROUND):
            pltpu.sync_copy(acc_a.at[pl.ds(r * WINDOW, WINDOW)],
                            shared.at[sid])
            plsc.subcore_barrier()
            pltpu.sync_copy(shared.at[:, pl.ds(sid * STRIPE, STRIPE)], rbuf)

            def rsum_body(j, carry):
                a = rbuf[0, pl.ds(j * L, L)]
                for t in range(1, NSUB):
                    a = a + rbuf[t, pl.ds(j * L, L)]
                tsum_buf[pl.ds(r * STRIPE + j * L, L)] = a
                return carry
            lax.fori_loop(0, spc, rsum_body, 0)
            plsc.subcore_barrier()

            pltpu.sync_copy(acc_q.at[pl.ds(r * WINDOW, WINDOW)],
                            shared.at[sid])
            plsc.subcore_barrier()
            pltpu.sync_copy(shared.at[:, pl.ds(sid * STRIPE, STRIPE)], rbuf)

            def loss_body(j, acc):
                q = rbuf[0, pl.ds(j * L, L)]
                for t in range(1, NSUB):
                    q = q + rbuf[t, pl.ds(j * L, L)]
                a = tsum_buf[pl.ds(r * STRIPE + j * L, L)]
                o = ones_buf[pl.ds(r * STRIPE + j * L, L)]
                s = _sqrt_pos(q)
                d1 = a - o
                d2 = s - o
                return acc + d1 * d1 + d2 * d2
            lv = lax.fori_loop(0, spc, loss_body, lv)
            if r != NROUND - 1:
                plsc.subcore_barrier()

        loss_buf[...] = lv
        wid = cid * NSUB + sid
        pltpu.sync_copy(loss_buf, out_hbm.at[wid])

    return k


def kernel(edge_index, score, target_edges, num_nodes):
    n_edges = score.shape[0]
    n_nodes = target_edges.shape[1]
    edge_flat = edge_index.reshape(-1)
    tgt_pad = jnp.pad(target_edges, ((0, 0), (0, N_PAD - n_nodes)),
                      mode="edge").reshape(-1)
    k = _build(n_edges)
    partials = k(edge_flat, score, tgt_pad)
    return jnp.sqrt(partials.sum() / (4.0 * num_nodes))
